# baseline (speedup 1.0000x reference)
import jax
import jax.numpy as jnp
from jax import lax
from jax.experimental import pallas as pl
from jax.experimental.pallas import tpu as pltpu

HIDDEN = 2048
NUM_EXPERTS = 16
TOP_K = 2

CHUNK = 256
NBUF = 8
NSPLIT = 2
NOUT = 4


def _router_body(x_hbm, wt_ref, logits_hbm, probs_hbm, w_hbm, i_hbm,
                 buf, sem, st_l, st_p, st_w, st_i, osem):
    n_chunks = x_hbm.shape[0] // CHUNK
    csz = HIDDEN // NSPLIT

    def start_copy(i, slot):
        for j in range(NSPLIT):
            pltpu.make_async_copy(
                x_hbm.at[pl.ds(i * CHUNK, CHUNK), pl.ds(j * csz, csz)],
                buf.at[slot, slice(None), pl.ds(j * csz, csz)],
                sem.at[slot, j],
            ).start()

    def wait_copy(slot):
        for j in range(NSPLIT):
            pltpu.make_async_copy(
                x_hbm.at[pl.ds(0, CHUNK), pl.ds(0, csz)],
                buf.at[slot, slice(None), pl.ds(j * csz, csz)],
                sem.at[slot, j],
            ).wait()

    def out_copies(i, oslot):
        row0 = i * CHUNK
        return [
            pltpu.make_async_copy(
                st_l.at[oslot], logits_hbm.at[pl.ds(row0, CHUNK), :],
                osem.at[oslot, 0]),
            pltpu.make_async_copy(
                st_p.at[oslot], probs_hbm.at[pl.ds(row0, CHUNK), :],
                osem.at[oslot, 1]),
            pltpu.make_async_copy(
                st_w.at[oslot], w_hbm.at[pl.ds(row0, CHUNK), :],
                osem.at[oslot, 2]),
            pltpu.make_async_copy(
                st_i.at[oslot], i_hbm.at[pl.ds(row0, CHUNK), :],
                osem.at[oslot, 3]),
        ]

    for s in range(NBUF):
        start_copy(s, s)

    wt = wt_ref[...]

    def chunk_body(i, _):
        slot = lax.rem(i, NBUF)
        oslot = lax.rem(i, NOUT)
        wait_copy(slot)
        logits = jax.lax.dot_general(
            buf[slot], wt, (((1,), (0,)), ((), ())),
            preferred_element_type=jnp.float32)

        @pl.when(i + NBUF < n_chunks)
        def _():
            start_copy(i + NBUF, slot)

        @pl.when(i >= NOUT)
        def _():
            for c in out_copies(i - NOUT, oslot):
                c.wait()

        m = jnp.max(logits, axis=-1, keepdims=True)
        e = jnp.exp(logits - m)
        ssum = jnp.sum(e, axis=-1, keepdims=True)
        probs = e / ssum

        iota = jax.lax.broadcasted_iota(jnp.int32, probs.shape, 1)
        p1 = jnp.max(probs, axis=-1, keepdims=True)
        i1 = jnp.argmax(probs, axis=-1, keepdims=True).astype(jnp.int32)
        masked = jnp.where(iota == i1, -jnp.inf, probs)
        p2 = jnp.max(masked, axis=-1, keepdims=True)
        i2 = jnp.argmax(masked, axis=-1, keepdims=True).astype(jnp.int32)
        denom = p1 + p2

        st_l[oslot] = logits
        st_p[oslot] = probs
        st_w[oslot] = jnp.concatenate([p1 / denom, p2 / denom], axis=-1)
        st_i[oslot] = jnp.concatenate([i1, i2], axis=-1)
        for c in out_copies(i, oslot):
            c.start()
        return 0

    lax.fori_loop(0, n_chunks, chunk_body, 0)
    for k in range(NOUT):
        i = n_chunks - NOUT + k
        for c in out_copies(i, lax.rem(jnp.int32(i), NOUT)):
            c.wait()


@jax.jit
def kernel(x, W):
    B, S, H = x.shape
    N = B * S
    x2 = x.reshape(N, H)
    wt = W.T

    logits, probs, weights, idx = pl.pallas_call(
        _router_body,
        in_specs=[
            pl.BlockSpec(memory_space=pl.ANY),
            pl.BlockSpec((H, NUM_EXPERTS), lambda: (0, 0)),
        ],
        out_specs=[
            pl.BlockSpec(memory_space=pl.ANY),
            pl.BlockSpec(memory_space=pl.ANY),
            pl.BlockSpec(memory_space=pl.ANY),
            pl.BlockSpec(memory_space=pl.ANY),
        ],
        out_shape=[
            jax.ShapeDtypeStruct((N, NUM_EXPERTS), jnp.float32),
            jax.ShapeDtypeStruct((N, NUM_EXPERTS), jnp.float32),
            jax.ShapeDtypeStruct((N, TOP_K), jnp.float32),
            jax.ShapeDtypeStruct((N, TOP_K), jnp.int32),
        ],
        scratch_shapes=[
            pltpu.VMEM((NBUF, CHUNK, HIDDEN), jnp.float32),
            pltpu.SemaphoreType.DMA((NBUF, NSPLIT)),
            pltpu.VMEM((NOUT, CHUNK, NUM_EXPERTS), jnp.float32),
            pltpu.VMEM((NOUT, CHUNK, NUM_EXPERTS), jnp.float32),
            pltpu.VMEM((NOUT, CHUNK, TOP_K), jnp.float32),
            pltpu.VMEM((NOUT, CHUNK, TOP_K), jnp.int32),
            pltpu.SemaphoreType.DMA((NOUT, 4)),
        ],
    )(x2, wt)

    routing_weights = weights.reshape(B, S, TOP_K)
    expert_indices = idx.reshape(B, S, TOP_K)
    return (routing_weights, expert_indices, logits, probs)
